# baseline (speedup 1.0000x reference)
import jax
import jax.numpy as jnp
from jax import lax
from jax.experimental import pallas as pl
from jax.experimental.pallas import tpu as pltpu

_B = 1024
_C = 100000
_W = 4096
_RG = 2
_RB = _B // _RG
_NBLK = (_C + _W - 1) // _W


def _loss_kernel(costh_ref, label_ref, out_ref, se_acc, g_acc):
    i, jb = pl.program_id(0), pl.program_id(1)

    @pl.when((i == 0) & (jb == 0))
    def _init():
        se_acc[...] = jnp.zeros_like(se_acc)
        g_acc[...] = jnp.zeros_like(g_acc)

    c = costh_ref[...]
    lrel = label_ref[...] - jb * _W
    is_lab = lax.broadcasted_iota(jnp.int32, (_RB, _W), 1) == lrel
    rows = pl.ds(i * _RB, _RB)
    g_acc[rows, :] += jnp.sum(jnp.where(is_lab, c, 0.0), axis=1, keepdims=True)

    @pl.when(jb < _NBLK - 1)
    def _main():
        se_acc[rows, :] += jnp.sum(jnp.exp(8.5 * c), axis=1, keepdims=True)

    @pl.when(jb == _NBLK - 1)
    def _maskblk():
        e = jnp.where(
            lax.broadcasted_iota(jnp.int32, (_RB, _W), 1) < (_C - jb * _W),
            jnp.exp(8.5 * c), 0.0)
        se_acc[rows, :] += jnp.sum(e, axis=1, keepdims=True)

    @pl.when((i == _RG - 1) & (jb == _NBLK - 1))
    def _fin():
        a_g = 8.5 * g_acc[...]
        picked = a_g - 2.25
        se = se_acc[...] - jnp.exp(a_g) + jnp.exp(picked)
        loss_i = jnp.log(se) - picked
        out_ref[...] = jnp.mean(loss_i, keepdims=True)


def kernel(costh, label):
    label2d = label.astype(jnp.int32).reshape(_B, 1)
    out = pl.pallas_call(
        _loss_kernel,
        grid=(_RG, _NBLK),
        in_specs=[
            pl.BlockSpec((_RB, _W), lambda i, j: (i, j)),
            pl.BlockSpec((_RB, 1), lambda i, j: (i, 0)),
        ],
        out_specs=pl.BlockSpec((1, 1), lambda i, j: (0, 0)),
        out_shape=jax.ShapeDtypeStruct((1, 1), jnp.float32),
        scratch_shapes=[
            pltpu.VMEM((_B, 1), jnp.float32),
            pltpu.VMEM((_B, 1), jnp.float32),
        ],
    )(costh, label2d)
    return out[0, 0]
